# baseline (speedup 1.0000x reference)
import functools

import jax
import jax.numpy as jnp
from jax import lax
from jax.experimental import pallas as pl
from jax.experimental.pallas import tpu as pltpu
from jax.experimental.pallas import tpu_sc as plsc

_K = 16
_MT = 256
_GQ = 128
_L = 16
_NLS = _GQ // _L
_CH = 256
_NW = 32
_SB = 64
_BLK = 16


def _dist_body(s_ref, q_ref, d_ref):
    s = s_ref[0]
    q = q_ref[0]
    r = lax.dot_general(
        s, q, (((1,), (1,)), ((), ())), preferred_element_type=jnp.float32
    )
    s2 = jnp.sum(s * s, axis=1, keepdims=True)
    q2 = jnp.sum(q * q, axis=1)[None, :]
    d_ref[0] = jnp.sqrt(jnp.clip(s2 + q2 - 2.0 * r, 0.0, None))


def _dist(support, query):
    b, n, c = support.shape
    m = query.shape[1]
    return pl.pallas_call(
        _dist_body,
        grid=(b, m // _MT),
        in_specs=[
            pl.BlockSpec((1, n, c), lambda bi, mi: (bi, 0, 0)),
            pl.BlockSpec((1, _MT, c), lambda bi, mi: (bi, mi, 0)),
        ],
        out_specs=pl.BlockSpec((1, n, _MT), lambda bi, mi: (bi, 0, mi)),
        out_shape=jax.ShapeDtypeStruct((b, n, m), jnp.float32),
    )(support, query)


def _make_sc_topk(b, n, m):
    ng = b * m // _GQ
    ngw = ng // _NW
    nch = n // _CH
    mg = m // _GQ
    mesh = plsc.VectorSubcoreMesh(core_axis_name="c", subcore_axis_name="s")

    @functools.partial(
        pl.kernel,
        mesh=mesh,
        out_type=[
            jax.ShapeDtypeStruct((ng, _NLS, _K, _L), jnp.float32),
            jax.ShapeDtypeStruct((ng, _NLS, _K, _L), jnp.int32),
        ],
        scratch_types=[
            pltpu.VMEM((_CH, _GQ), jnp.float32),
            pltpu.VMEM((_CH, _GQ), jnp.float32),
            pltpu.VMEM((_NLS, _K, _L), jnp.float32),
            pltpu.VMEM((_NLS, _K, _L), jnp.int32),
            pltpu.SemaphoreType.DMA,
            pltpu.SemaphoreType.DMA,
        ],
    )
    def sc_topk(d_hbm, vals_hbm, idxs_hbm, buf0, buf1, sbuf, sibuf, sem0, sem1):
        wid = lax.axis_index("s") * 2 + lax.axis_index("c")
        ii = lax.iota(jnp.int32, _L)
        inf_v = jnp.full((_L,), jnp.inf, jnp.float32)
        zero_v = jnp.zeros((_L,), jnp.int32)

        def lane_min(x):
            x = jnp.minimum(x, x.at[ii ^ 8].get(mode="promise_in_bounds"))
            x = jnp.minimum(x, x.at[ii ^ 4].get(mode="promise_in_bounds"))
            x = jnp.minimum(x, x.at[ii ^ 2].get(mode="promise_in_bounds"))
            x = jnp.minimum(x, x.at[ii ^ 1].get(mode="promise_in_bounds"))
            return x[0]

        def process_rows(buf, base):
            def ls_body(j, _):
                def sb_body(sb, _, j=j):
                    def blk_body(ib, _, j=j, sb=sb):
                        t = sbuf[j, _K - 1]
                        i0 = sb * _SB + ib * _BLK
                        vs = [buf[i0 + u, pl.ds(j * _L, _L)] for u in range(_BLK)]
                        bm = vs[0]
                        for u in range(1, _BLK):
                            bm = jnp.minimum(bm, vs[u])

                        @pl.when(lane_min(bm - t) < 0.0)
                        def _rescan():
                            for u in range(_BLK):
                                v = vs[u]

                                @pl.when(lane_min(v - t) < 0.0)
                                def _insert(v=v, u=u):
                                    c = v
                                    ci = zero_v + (base + i0 + u)
                                    for k in range(_K):
                                        s_k = sbuf[j, k]
                                        si_k = sibuf[j, k]
                                        swap = c < s_k
                                        sbuf[j, k] = jnp.where(swap, c, s_k)
                                        sibuf[j, k] = jnp.where(swap, ci, si_k)
                                        c2 = jnp.where(swap, s_k, c)
                                        ci2 = jnp.where(swap, si_k, ci)
                                        c, ci = c2, ci2

                        return 0

                    lax.fori_loop(0, _SB // _BLK, blk_body, 0, unroll=2)
                    return 0

                lax.fori_loop(0, _CH // _SB, sb_body, 0)
                return 0

            lax.fori_loop(0, _NLS, ls_body, 0)

        for gi in range(ngw):
            g = wid * ngw + gi
            bb = g // mg
            mm = (g % mg) * _GQ

            def init_body(j, _):
                for k in range(_K):
                    sbuf[j, k] = inf_v
                    sibuf[j, k] = zero_v
                return 0

            lax.fori_loop(0, _NLS, init_body, 0)

            pltpu.async_copy(
                d_hbm.at[bb, pl.ds(0, _CH), pl.ds(mm, _GQ)], buf0, sem0
            )

            def chunk_body(c2, _, bb=bb, mm=mm):
                ch0 = c2 * 2
                cp1 = pltpu.async_copy(
                    d_hbm.at[bb, pl.ds((ch0 + 1) * _CH, _CH), pl.ds(mm, _GQ)],
                    buf1,
                    sem1,
                )
                pltpu.make_async_copy(
                    d_hbm.at[bb, pl.ds(0, _CH), pl.ds(mm, _GQ)], buf0, sem0
                ).wait()
                process_rows(buf0, ch0 * _CH)

                @pl.when(ch0 + 2 < nch)
                def _prefetch_even():
                    pltpu.async_copy(
                        d_hbm.at[bb, pl.ds((ch0 + 2) * _CH, _CH), pl.ds(mm, _GQ)],
                        buf0,
                        sem0,
                    )

                cp1.wait()
                process_rows(buf1, (ch0 + 1) * _CH)
                return 0

            lax.fori_loop(0, nch // 2, chunk_body, 0)

            pltpu.sync_copy(sbuf, vals_hbm.at[g])
            pltpu.sync_copy(sibuf, idxs_hbm.at[g])

    return sc_topk


@jax.jit
def kernel(support, query):
    b, n, c = support.shape
    m = query.shape[1]
    d = _dist(support, query)
    vals_g, idxs_g = _make_sc_topk(b, n, m)(d)
    mg = m // _GQ
    vals_t = vals_g.reshape(b, mg, _NLS, _K, _L)
    idxs_t = idxs_g.reshape(b, mg, _NLS, _K, _L)
    vals = vals_t.transpose(0, 3, 1, 2, 4).reshape(b, _K, m)
    idxs = idxs_t.transpose(0, 1, 2, 4, 3).reshape(b, m, _K)
    return vals, idxs


# speedup vs baseline: 3.7586x; 3.7586x over previous
import functools

import jax
import jax.numpy as jnp
from jax import lax
from jax.experimental import pallas as pl
from jax.experimental.pallas import tpu as pltpu
from jax.experimental.pallas import tpu_sc as plsc

_K = 16
_MT = 256
_GQ = 128
_L = 16
_NLS = _GQ // _L
_CH = 256
_NW = 32
_SB = 64
_BLK = 8


def _dist_body(s_ref, q_ref, d_ref):
    s = s_ref[0]
    q = q_ref[0]
    r = lax.dot_general(
        s, q, (((1,), (1,)), ((), ())), preferred_element_type=jnp.float32
    )
    s2 = jnp.sum(s * s, axis=1, keepdims=True)
    q2 = jnp.sum(q * q, axis=1)[None, :]
    d_ref[0] = jnp.sqrt(jnp.clip(s2 + q2 - 2.0 * r, 0.0, None))


def _dist(support, query):
    b, n, c = support.shape
    m = query.shape[1]
    return pl.pallas_call(
        _dist_body,
        grid=(b, m // _MT),
        in_specs=[
            pl.BlockSpec((1, n, c), lambda bi, mi: (bi, 0, 0)),
            pl.BlockSpec((1, _MT, c), lambda bi, mi: (bi, mi, 0)),
        ],
        out_specs=pl.BlockSpec((1, n, _MT), lambda bi, mi: (bi, 0, mi)),
        out_shape=jax.ShapeDtypeStruct((b, n, m), jnp.float32),
    )(support, query)


def _make_sc_topk(b, n, m):
    ng = b * m // _GQ
    ngw = ng // _NW
    nch = n // _CH
    mg = m // _GQ
    mesh = plsc.VectorSubcoreMesh(core_axis_name="c", subcore_axis_name="s")

    @functools.partial(
        pl.kernel,
        mesh=mesh,
        out_type=[
            jax.ShapeDtypeStruct((ng, _NLS, _K, _L), jnp.float32),
            jax.ShapeDtypeStruct((ng, _NLS, _K, _L), jnp.int32),
        ],
        scratch_types=[
            pltpu.VMEM((_CH, _GQ), jnp.float32),
            pltpu.VMEM((_CH, _GQ), jnp.float32),
            pltpu.VMEM((_NLS, _K, _L), jnp.float32),
            pltpu.VMEM((_NLS, _K, _L), jnp.int32),
            pltpu.SemaphoreType.DMA,
            pltpu.SemaphoreType.DMA,
        ],
    )
    def sc_topk(d_hbm, vals_hbm, idxs_hbm, buf0, buf1, sbuf, sibuf, sem0, sem1):
        wid = lax.axis_index("s") * 2 + lax.axis_index("c")
        ii = lax.iota(jnp.int32, _L)
        inf_v = jnp.full((_L,), jnp.inf, jnp.float32)
        zero_v = jnp.zeros((_L,), jnp.int32)

        def lane_min(x):
            x = jnp.minimum(x, x.at[ii ^ 8].get(mode="promise_in_bounds"))
            x = jnp.minimum(x, x.at[ii ^ 4].get(mode="promise_in_bounds"))
            x = jnp.minimum(x, x.at[ii ^ 2].get(mode="promise_in_bounds"))
            x = jnp.minimum(x, x.at[ii ^ 1].get(mode="promise_in_bounds"))
            return x[0]

        def process_rows(buf, base):
            def ls_body(j, _):
                def sb_body(sb, _, j=j):
                    def blk_body(ib, _, j=j, sb=sb):
                        t = sbuf[j, _K - 1]
                        i0 = sb * _SB + ib * _BLK
                        vs = [buf[i0 + u, pl.ds(j * _L, _L)] for u in range(_BLK)]
                        bm = vs[0]
                        for u in range(1, _BLK):
                            bm = jnp.minimum(bm, vs[u])

                        @pl.when(lane_min(bm - t) < 0.0)
                        def _rescan():
                            for u in range(_BLK):
                                v = vs[u]

                                @pl.when(lane_min(v - t) < 0.0)
                                def _insert(v=v, u=u):
                                    c = v
                                    ci = zero_v + (base + i0 + u)
                                    for k in range(_K):
                                        s_k = sbuf[j, k]
                                        si_k = sibuf[j, k]
                                        swap = c < s_k
                                        sbuf[j, k] = jnp.where(swap, c, s_k)
                                        sibuf[j, k] = jnp.where(swap, ci, si_k)
                                        c2 = jnp.where(swap, s_k, c)
                                        ci2 = jnp.where(swap, si_k, ci)
                                        c, ci = c2, ci2

                        return 0

                    lax.fori_loop(0, _SB // _BLK, blk_body, 0, unroll=2)
                    return 0

                lax.fori_loop(0, _CH // _SB, sb_body, 0)
                return 0

            lax.fori_loop(0, _NLS, ls_body, 0)

        for gi in range(ngw):
            g = wid * ngw + gi
            bb = g // mg
            mm = (g % mg) * _GQ

            def init_body(j, _):
                for k in range(_K):
                    sbuf[j, k] = inf_v
                    sibuf[j, k] = zero_v
                return 0

            lax.fori_loop(0, _NLS, init_body, 0)

            pltpu.async_copy(
                d_hbm.at[bb, pl.ds(0, _CH), pl.ds(mm, _GQ)], buf0, sem0
            )

            def chunk_body(c2, _, bb=bb, mm=mm):
                ch0 = c2 * 2
                cp1 = pltpu.async_copy(
                    d_hbm.at[bb, pl.ds((ch0 + 1) * _CH, _CH), pl.ds(mm, _GQ)],
                    buf1,
                    sem1,
                )
                pltpu.make_async_copy(
                    d_hbm.at[bb, pl.ds(0, _CH), pl.ds(mm, _GQ)], buf0, sem0
                ).wait()
                process_rows(buf0, ch0 * _CH)

                @pl.when(ch0 + 2 < nch)
                def _prefetch_even():
                    pltpu.async_copy(
                        d_hbm.at[bb, pl.ds((ch0 + 2) * _CH, _CH), pl.ds(mm, _GQ)],
                        buf0,
                        sem0,
                    )

                cp1.wait()
                process_rows(buf1, (ch0 + 1) * _CH)
                return 0

            lax.fori_loop(0, nch // 2, chunk_body, 0)

            pltpu.sync_copy(sbuf, vals_hbm.at[g])
            pltpu.sync_copy(sibuf, idxs_hbm.at[g])

    return sc_topk


@jax.jit
def kernel(support, query):
    b, n, c = support.shape
    m = query.shape[1]
    d = _dist(support, query)
    vals_g, idxs_g = _make_sc_topk(b, n, m)(d)
    mg = m // _GQ
    vals_t = vals_g.reshape(b, mg, _NLS, _K, _L)
    idxs_t = idxs_g.reshape(b, mg, _NLS, _K, _L)
    vals = vals_t.transpose(0, 3, 1, 2, 4).reshape(b, _K, m)
    idxs = idxs_t.transpose(0, 1, 2, 4, 3).reshape(b, m, _K)
    return vals, idxs
